# Initial kernel scaffold; baseline (speedup 1.0000x reference)
import jax
import jax.numpy as jnp
from jax.experimental import pallas as pl


def kernel(features, edge_index, batch, W1, b1, g1, be1, W2, b2, g2, be2, Wm1, bm1, gm, bem, Wm2, bm2):
    raise NotImplementedError("write your pallas kernel here")


# speedup vs baseline: 15.6829x; 15.6829x over previous
import functools

import jax
import jax.numpy as jnp
from jax import lax
from jax.experimental import pallas as pl
from jax.experimental.pallas import tpu as pltpu
from jax.experimental.pallas import tpu_sc as plsc

N = 10000
NP = 10240
D = 128
E = 320000
G = 100
NC = 2
NS = 16
NW = NC * NS
EPT = E // NW
CH = 80
NCHUNK = EPT // CH
RPT = NP // NS
EPS = 1e-5

_mesh = plsc.VectorSubcoreMesh(
    core_axis_name="c", subcore_axis_name="s", num_cores=NC, num_subcores=NS)

_HIGH = jax.lax.Precision.HIGHEST


def _dot(a, b):
    return jnp.dot(a, b, preferred_element_type=jnp.float32, precision=_HIGH)


def _deg_body(dst_h, zer_h, one_h, out_h, dall, ones_v, acc):
    c = lax.axis_index("c")
    s = lax.axis_index("s")
    wid = c * NS + s

    @pl.when(s == 0)
    def _init():
        pltpu.sync_copy(zer_h, acc)

    pltpu.sync_copy(one_h, ones_v)
    pltpu.sync_copy(dst_h.at[wid], dall)
    plsc.subcore_barrier()

    def body(j, car):
        pltpu.sync_copy(ones_v, acc.at[dall.at[j]], add=True)
        return car

    lax.fori_loop(0, NCHUNK, body, 0)
    plsc.subcore_barrier()
    off = pl.multiple_of(s * RPT, 8)
    pltpu.sync_copy(acc.at[pl.ds(off, RPT)], out_h.at[c, pl.ds(off, RPT)])


@functools.partial(
    pl.kernel,
    out_type=jax.ShapeDtypeStruct((NC, NP, D), jnp.float32),
    mesh=_mesh,
    scratch_types=[
        pltpu.VMEM((NCHUNK, CH), jnp.int32),
        pltpu.VMEM((CH, D), jnp.float32),
        pltpu.VMEM_SHARED((NP, D), jnp.float32),
    ],
)
def _sc_deg(dst_h, zer_h, one_h, out_h, dall, ones_v, acc):
    _deg_body(dst_h, zer_h, one_h, out_h, dall, ones_v, acc)


def _spmm_body(y_h, src_h, dst_h, zer_h, out_h, sall, dall, rows, acc):
    c = lax.axis_index("c")
    s = lax.axis_index("s")
    wid = c * NS + s

    @pl.when(s == 0)
    def _init():
        pltpu.sync_copy(zer_h, acc)

    pltpu.sync_copy(src_h.at[wid], sall)
    pltpu.sync_copy(dst_h.at[wid], dall)
    plsc.subcore_barrier()

    def body(j, car):
        pltpu.sync_copy(y_h.at[sall.at[j]], rows)
        pltpu.sync_copy(rows, acc.at[dall.at[j]], add=True)
        return car

    lax.fori_loop(0, NCHUNK, body, 0)
    plsc.subcore_barrier()
    off = pl.multiple_of(s * RPT, 8)
    pltpu.sync_copy(acc.at[pl.ds(off, RPT)], out_h.at[c, pl.ds(off, RPT)])


@functools.partial(
    pl.kernel,
    out_type=jax.ShapeDtypeStruct((NC, NP, D), jnp.float32),
    mesh=_mesh,
    scratch_types=[
        pltpu.VMEM((NCHUNK, CH), jnp.int32),
        pltpu.VMEM((NCHUNK, CH), jnp.int32),
        pltpu.VMEM((CH, D), jnp.float32),
        pltpu.VMEM_SHARED((NP, D), jnp.float32),
    ],
)
def _sc_spmm(y_h, src_h, dst_h, zer_h, out_h, sall, dall, rows, acc):
    _spmm_body(y_h, src_h, dst_h, zer_h, out_h, sall, dall, rows, acc)


def _lin1_body(x_ref, w_ref, d0_ref, d1_ref, y_ref, dinv_ref):
    deg = d0_ref[...] + d1_ref[...] + 1.0
    dinv = lax.rsqrt(jnp.maximum(deg, 1.0))
    y_ref[...] = _dot(x_ref[...], w_ref[...]) * dinv
    dinv_ref[...] = dinv


def _bn_relu(t, g, be):
    m = jnp.mean(t, axis=0, keepdims=True)
    tc = t - m
    v = jnp.mean(tc * tc, axis=0, keepdims=True)
    return jnp.maximum(tc * lax.rsqrt(v + EPS) * g + be, 0.0)


def _mid_body(p0_ref, p1_ref, y_ref, dinv_ref, g_ref, be_ref, w_ref, out_ref):
    dinv = dinv_ref[...]
    t = (p0_ref[...] + p1_ref[...] + y_ref[...]) * dinv
    t = _bn_relu(t, g_ref[...], be_ref[...])
    out_ref[...] = _dot(t, w_ref[...]) * dinv


def _head_body(p0_ref, p1_ref, y_ref, dinv_ref, g_ref, be_ref, bat_ref,
               wm1_ref, gm_ref, bem_ref, wm2_ref, bm2_ref, out_ref):
    t = (p0_ref[...] + p1_ref[...] + y_ref[...]) * dinv_ref[...]
    h = _bn_relu(t, g_ref[...], be_ref[...])
    gid = lax.broadcasted_iota(jnp.int32, (G, N), 0)
    m = (gid == bat_ref[...]).astype(jnp.float32)
    ssum = _dot(m, h)
    cnt = jnp.sum(m, axis=1, keepdims=True)
    pooled = ssum / jnp.maximum(cnt, 1.0)
    q = _bn_relu(_dot(pooled, wm1_ref[...]), gm_ref[...], bem_ref[...])
    logits = _dot(q, wm2_ref[...]) + bm2_ref[...]
    out_ref[...] = 1.0 / (1.0 + jnp.exp(-logits))


def _tc_call(body, out_shape):
    return pl.pallas_call(body, out_shape=out_shape)


def kernel(features, edge_index, batch, W1, b1, g1, be1, W2, b2, g2, be2,
           Wm1, bm1, gm, bem, Wm2, bm2):
    x = features.reshape(N, D)
    src = edge_index[0].reshape(NW, NCHUNK, CH)
    dst = edge_index[1].reshape(NW, NCHUNK, CH)
    zeros_d = jnp.zeros((NP, D), jnp.float32)
    ones_d = jnp.ones((CH, D), jnp.float32)

    degp = _sc_deg(dst, zeros_d, ones_d)

    y1, dinv = _tc_call(
        _lin1_body,
        (jax.ShapeDtypeStruct((N, D), jnp.float32),
         jax.ShapeDtypeStruct((N, 1), jnp.float32)),
    )(x, W1, degp[0, :N, 0:1], degp[1, :N, 0:1])

    p1 = _sc_spmm(y1, src, dst, zeros_d)

    y2 = _tc_call(_mid_body, jax.ShapeDtypeStruct((N, D), jnp.float32))(
        p1[0, :N], p1[1, :N], y1, dinv, g1.reshape(1, D), be1.reshape(1, D),
        W2)

    p2 = _sc_spmm(y2, src, dst, zeros_d)

    out = _tc_call(_head_body, jax.ShapeDtypeStruct((G, 16), jnp.float32))(
        p2[0, :N], p2[1, :N], y2, dinv, g2.reshape(1, D), be2.reshape(1, D),
        batch.reshape(1, N), Wm1, gm.reshape(1, D), bem.reshape(1, D),
        Wm2, bm2.reshape(1, 16))
    return out
